# baseline (speedup 1.0000x reference)
import functools

import jax
import jax.numpy as jnp
from jax import lax
from jax.experimental import pallas as pl
from jax.experimental.pallas import tpu as pltpu
from jax.experimental.pallas import tpu_sc as plsc

_N = 10000
_E = 320000
_D = 128
_G = 128

_NC = 2
_NS = 16
_NW = _NC * _NS
_EW = _E // _NW
_K = 80
_T = _EW // _K
_RPT = 632
_NP = _NS * _RPT

_BLK = 2000
_NBLK = _N // _BLK


def _sc_agg_body(x_hbm, src_hbm, dst_hbm, zeros_hbm, out_hbm,
                 acc, sidx, di0, di1, r0, r1, gs0, gs1, ss0, ss1, is0, is1):
    r = [r0, r1]
    di = [di0, di1]
    gs = [gs0, gs1]
    ss = [ss0, ss1]
    isem = [is0, is1]
    c = lax.axis_index("c")
    s = lax.axis_index("s")
    w = s * _NC + c

    pltpu.sync_copy(zeros_hbm, acc.at[pl.ds(s * _RPT, _RPT)])
    pltpu.sync_copy(src_hbm.at[w], sidx)

    def didx_start(t, j):
        pltpu.async_copy(dst_hbm.at[w, t], di[j], isem[j])

    def didx_wait(t, j):
        pltpu.make_async_copy(dst_hbm.at[w, t], di[j], isem[j]).wait()

    def gather_start(t, j):
        pltpu.async_copy(x_hbm.at[sidx.at[t]], r[j], gs[j])

    def gather_wait(t, j):
        pltpu.make_async_copy(x_hbm.at[sidx.at[t]], r[j], gs[j]).wait()

    def scatter_start(j):
        pltpu.async_copy(r[j], acc.at[di[j]], ss[j], add=True)

    def scatter_wait(j):
        pltpu.make_async_copy(r[j], acc.at[di[j]], ss[j]).wait()

    plsc.subcore_barrier()

    def _start(i, _):
        didx_start(i, 0)
        gather_start(i, 0)
        didx_wait(i, 0)
        gather_wait(i, 0)
        scatter_start(0)
        didx_start(i + 1, 1)
        gather_start(i + 1, 1)
        return 0
    lax.fori_loop(0, jnp.int32(1), _start, 0)

    def _pair(i, _):
        t = 2 * i + 1
        gather_wait(t, 1)
        didx_wait(t, 1)
        scatter_start(1)
        scatter_wait(0)
        didx_start(t + 1, 0)
        gather_start(t + 1, 0)
        gather_wait(t + 1, 0)
        didx_wait(t + 1, 0)
        scatter_start(0)
        scatter_wait(1)

        @pl.when(t + 2 < _T)
        def _():
            didx_start(t + 2, 1)
            gather_start(t + 2, 1)
        return 0
    lax.fori_loop(0, (_T - 1) // 2, _pair, 0)

    scatter_wait(0)

    plsc.subcore_barrier()

    pltpu.sync_copy(acc.at[pl.ds(s * _RPT, _RPT)],
                    out_hbm.at[c, pl.ds(s * _RPT, _RPT)])


_sc_agg = functools.partial(
    pl.kernel,
    out_type=jax.ShapeDtypeStruct((_NC, _NP, _D), jnp.float32),
    mesh=plsc.VectorSubcoreMesh(core_axis_name="c", subcore_axis_name="s"),
    scratch_types=[
        pltpu.VMEM_SHARED((_NP, _D), jnp.float32),
        pltpu.VMEM((_T, _K), jnp.int32),
        pltpu.VMEM((_K,), jnp.int32),
        pltpu.VMEM((_K,), jnp.int32),
        pltpu.VMEM((_K, _D), jnp.float32),
        pltpu.VMEM((_K, _D), jnp.float32),
        pltpu.SemaphoreType.DMA,
        pltpu.SemaphoreType.DMA,
        pltpu.SemaphoreType.DMA,
        pltpu.SemaphoreType.DMA,
        pltpu.SemaphoreType.DMA,
        pltpu.SemaphoreType.DMA,
    ],
)(_sc_agg_body)


def _stats_accum(i, y, st_ref):
    s1 = jnp.sum(y, axis=0, keepdims=True)
    s2 = jnp.sum(y * y, axis=0, keepdims=True)
    upd = jnp.concatenate([s1, s2, jnp.zeros((6, _D), jnp.float32)], axis=0)

    @pl.when(i == 0)
    def _():
        st_ref[...] = upd

    @pl.when(i > 0)
    def _():
        st_ref[...] += upd


def _bn_affine(st, g, be):
    m = st[0:1, :] * (1.0 / _N)
    v = st[1:2, :] * (1.0 / _N) - m * m
    a = lax.rsqrt(v + 1e-5) * g
    return a, be - m * a


def _passA_body(h_ref, a0_ref, a1_ref, w_ref, b_ref, y_ref, st_ref):
    i = pl.program_id(0)
    u = h_ref[...] + a0_ref[0] + a1_ref[0]
    y = jnp.dot(u, w_ref[...], preferred_element_type=jnp.float32) + b_ref[...]
    y_ref[...] = y
    _stats_accum(i, y, st_ref)


def _passB_body(y1_ref, st1_ref, g_ref, be_ref, w_ref, b_ref, y2_ref, st2_ref):
    i = pl.program_id(0)
    a, cc = _bn_affine(st1_ref[...], g_ref[...], be_ref[...])
    z = jnp.maximum(y1_ref[...] * a + cc, 0.0)
    y2 = jnp.dot(z, w_ref[...], preferred_element_type=jnp.float32) + b_ref[...]
    y2_ref[...] = y2
    _stats_accum(i, y2, st2_ref)


def _passC_body(y2_ref, st_ref, g_ref, be_ref, h_ref):
    a, cc = _bn_affine(st_ref[...], g_ref[...], be_ref[...])
    h_ref[...] = jnp.maximum(y2_ref[...] * a + cc, 0.0)


def _passC2_body(y2_ref, st_ref, g_ref, be_ref, bt_ref, lw_ref, lb_ref,
                 out_ref, pacc):
    i = pl.program_id(0)
    a, cc = _bn_affine(st_ref[...], g_ref[...], be_ref[...])
    h = jnp.maximum(y2_ref[...] * a + cc, 0.0)
    bt = bt_ref[0, 0, :]
    onehot = (bt[:, None] == lax.broadcasted_iota(jnp.int32, (_BLK, _G), 1)
              ).astype(jnp.float32)
    p = lax.dot_general(onehot, h, (((0,), (0,)), ((), ())),
                        preferred_element_type=jnp.float32)

    @pl.when(i == 0)
    def _():
        pacc[...] = p

    @pl.when(i > 0)
    def _():
        pacc[...] += p

    @pl.when(i == _NBLK - 1)
    def _():
        out_ref[...] = (jnp.dot(pacc[...], lw_ref[...],
                                preferred_element_type=jnp.float32)
                        + lb_ref[...])


_blk2 = pl.BlockSpec((_BLK, _D), lambda i: (i, 0))
_full_st = pl.BlockSpec((8, _D), lambda i: (0, 0))
_full_w = pl.BlockSpec((_D, _D), lambda i: (0, 0))
_full_v = pl.BlockSpec((1, _D), lambda i: (0, 0))

_passA = pl.pallas_call(
    _passA_body,
    grid=(_NBLK,),
    in_specs=[
        _blk2,
        pl.BlockSpec((1, _BLK, _D), lambda i: (0, i, 0)),
        pl.BlockSpec((1, _BLK, _D), lambda i: (1, i, 0)),
        _full_w, _full_v,
    ],
    out_specs=[_blk2, _full_st],
    out_shape=[jax.ShapeDtypeStruct((_N, _D), jnp.float32),
               jax.ShapeDtypeStruct((8, _D), jnp.float32)],
)

_passB = pl.pallas_call(
    _passB_body,
    grid=(_NBLK,),
    in_specs=[_blk2, _full_st, _full_v, _full_v, _full_w, _full_v],
    out_specs=[_blk2, _full_st],
    out_shape=[jax.ShapeDtypeStruct((_N, _D), jnp.float32),
               jax.ShapeDtypeStruct((8, _D), jnp.float32)],
)

_passC = pl.pallas_call(
    _passC_body,
    grid=(_NBLK,),
    in_specs=[_blk2, _full_st, _full_v, _full_v],
    out_specs=_blk2,
    out_shape=jax.ShapeDtypeStruct((_N, _D), jnp.float32),
)

_passC2 = pl.pallas_call(
    _passC2_body,
    grid=(_NBLK,),
    in_specs=[
        _blk2, _full_st, _full_v, _full_v,
        pl.BlockSpec((1, 1, _BLK), lambda i: (i, 0, 0)),
        pl.BlockSpec((_D, 2 * _D), lambda i: (0, 0)),
        pl.BlockSpec((1, 2 * _D), lambda i: (0, 0)),
    ],
    out_specs=pl.BlockSpec((_G, 2 * _D), lambda i: (0, 0)),
    out_shape=jax.ShapeDtypeStruct((_G, 2 * _D), jnp.float32),
    scratch_shapes=[pltpu.VMEM((_G, _D), jnp.float32)],
)


def kernel(x, edge_index, batch,
           W0a, b0a, g0a, be0a, W0b, b0b, g0b, be0b,
           W1a, b1a, g1a, be1a, W1b, b1b, g1b, be1b,
           W2a, b2a, g2a, be2a, W2b, b2b, g2b, be2b,
           linW, linb):
    src3 = edge_index[0].astype(jnp.int32).reshape(_NW, _T, _K)
    dst3 = edge_index[1].astype(jnp.int32).reshape(_NW, _T, _K)
    bt3 = batch.astype(jnp.int32).reshape(_NBLK, 1, _BLK)

    p = {
        "W0a": W0a, "b0a": b0a, "g0a": g0a, "be0a": be0a,
        "W0b": W0b, "b0b": b0b, "g0b": g0b, "be0b": be0b,
        "W1a": W1a, "b1a": b1a, "g1a": g1a, "be1a": be1a,
        "W1b": W1b, "b1b": b1b, "g1b": g1b, "be1b": be1b,
        "W2a": W2a, "b2a": b2a, "g2a": g2a, "be2a": be2a,
        "W2b": W2b, "b2b": b2b, "g2b": g2b, "be2b": be2b,
    }

    def row(v):
        return v.reshape(1, -1)

    zc = jnp.zeros((_RPT, _D), jnp.float32)
    h = x
    for l in range(3):
        agg = _sc_agg(h, src3, dst3, zc)
        y1, s1 = _passA(h, agg, agg, p[f"W{l}a"], row(p[f"b{l}a"]))
        y2, s2 = _passB(y1, s1, row(p[f"g{l}a"]), row(p[f"be{l}a"]),
                        p[f"W{l}b"], row(p[f"b{l}b"]))
        if l < 2:
            h = _passC(y2, s2, row(p[f"g{l}b"]), row(p[f"be{l}b"]))
        else:
            out = _passC2(y2, s2, row(p[f"g{l}b"]), row(p[f"be{l}b"]),
                          bt3, linW, row(linb))
    return out


# speedup vs baseline: 1.0560x; 1.0560x over previous
import functools

import jax
import jax.numpy as jnp
from jax import lax
from jax.experimental import pallas as pl
from jax.experimental.pallas import tpu as pltpu
from jax.experimental.pallas import tpu_sc as plsc

_N = 10000
_E = 320000
_D = 128
_G = 128

_NC = 2
_NS = 16
_NW = _NC * _NS
_EW = _E // _NW
_K = 80
_T = _EW // _K
_RPT = 632
_NP = _NS * _RPT

_BLK = 2000
_NBLK = _N // _BLK


def _sc_agg_body(x_hbm, src_hbm, dst_hbm, zeros_hbm, out_hbm,
                 acc, sidx, di0, di1, r0, r1, gs0, gs1, ss0, ss1, is0, is1):
    r = [r0, r1]
    di = [di0, di1]
    gs = [gs0, gs1]
    ss = [ss0, ss1]
    isem = [is0, is1]
    c = lax.axis_index("c")
    s = lax.axis_index("s")
    w = s * _NC + c

    pltpu.sync_copy(zeros_hbm, acc.at[pl.ds(s * _RPT, _RPT)])
    pltpu.sync_copy(src_hbm.at[w], sidx)

    def didx_start(t, j):
        pltpu.async_copy(dst_hbm.at[w, t], di[j], isem[j])

    def didx_wait(t, j):
        pltpu.make_async_copy(dst_hbm.at[w, t], di[j], isem[j]).wait()

    def gather_start(t, j):
        pltpu.async_copy(x_hbm.at[sidx.at[t]], r[j], gs[j])

    def gather_wait(t, j):
        pltpu.make_async_copy(x_hbm.at[sidx.at[t]], r[j], gs[j]).wait()

    def scatter_start(j):
        pltpu.async_copy(r[j], acc.at[di[j]], ss[j], add=True)

    def scatter_wait(j):
        pltpu.make_async_copy(r[j], acc.at[di[j]], ss[j]).wait()

    plsc.subcore_barrier()

    def _start(i, _):
        didx_start(i, 0)
        gather_start(i, 0)
        didx_wait(i, 0)
        gather_wait(i, 0)
        scatter_start(0)
        didx_start(i + 1, 1)
        gather_start(i + 1, 1)
        return 0
    lax.fori_loop(0, jnp.int32(1), _start, 0)

    def _pair(i, _):
        t = 2 * i + 1
        gather_wait(t, 1)
        didx_wait(t, 1)
        scatter_start(1)
        scatter_wait(0)
        didx_start(t + 1, 0)
        gather_start(t + 1, 0)
        gather_wait(t + 1, 0)
        didx_wait(t + 1, 0)
        scatter_start(0)
        scatter_wait(1)

        @pl.when(t + 2 < _T)
        def _():
            didx_start(t + 2, 1)
            gather_start(t + 2, 1)
        return 0
    lax.fori_loop(0, (_T - 1) // 2, _pair, 0)

    scatter_wait(0)

    plsc.subcore_barrier()

    pltpu.sync_copy(acc.at[pl.ds(s * _RPT, _RPT)],
                    out_hbm.at[c, pl.ds(s * _RPT, _RPT)])


_sc_agg = functools.partial(
    pl.kernel,
    out_type=jax.ShapeDtypeStruct((_NC, _NP, _D), jnp.float32),
    mesh=plsc.VectorSubcoreMesh(core_axis_name="c", subcore_axis_name="s"),
    scratch_types=[
        pltpu.VMEM_SHARED((_NP, _D), jnp.float32),
        pltpu.VMEM((_T, _K), jnp.int32),
        pltpu.VMEM((_K,), jnp.int32),
        pltpu.VMEM((_K,), jnp.int32),
        pltpu.VMEM((_K, _D), jnp.float32),
        pltpu.VMEM((_K, _D), jnp.float32),
        pltpu.SemaphoreType.DMA,
        pltpu.SemaphoreType.DMA,
        pltpu.SemaphoreType.DMA,
        pltpu.SemaphoreType.DMA,
        pltpu.SemaphoreType.DMA,
        pltpu.SemaphoreType.DMA,
    ],
)(_sc_agg_body)


def _bn_affine(st, g, be):
    m = st[0:1, :] * (1.0 / _N)
    v = st[1:2, :] * (1.0 / _N) - m * m
    a = lax.rsqrt(v + 1e-5) * g
    return a, be - m * a


def _acc_stats(i, y, st):
    s1 = jnp.sum(y, axis=0, keepdims=True)
    s2 = jnp.sum(y * y, axis=0, keepdims=True)
    upd = jnp.concatenate([s1, s2, jnp.zeros((6, _D), jnp.float32)], axis=0)

    @pl.when(i == 0)
    def _():
        st[...] = upd

    @pl.when(i > 0)
    def _():
        st[...] += upd


def _layer_common(p, i, h_ref, a0_ref, a1_ref, wa_ref, ba_ref, ga_ref,
                  bea_ref, wb_ref, bb_ref, y1s, y2s, st1, st2):
    @pl.when(p == 0)
    def _():
        u = h_ref[...] + a0_ref[0] + a1_ref[0]
        y = jnp.dot(u, wa_ref[...], preferred_element_type=jnp.float32) + ba_ref[...]
        y1s[pl.ds(i * _BLK, _BLK), :] = y
        _acc_stats(i, y, st1)

    @pl.when(p == 1)
    def _():
        a, cc = _bn_affine(st1[...], ga_ref[...], bea_ref[...])
        z = jnp.maximum(y1s[pl.ds(i * _BLK, _BLK), :] * a + cc, 0.0)
        y2 = jnp.dot(z, wb_ref[...], preferred_element_type=jnp.float32) + bb_ref[...]
        y2s[pl.ds(i * _BLK, _BLK), :] = y2
        _acc_stats(i, y2, st2)


def _layer_body(h_ref, a0_ref, a1_ref, wa_ref, ba_ref, ga_ref, bea_ref,
                wb_ref, bb_ref, gb_ref, beb_ref, out_ref, y1s, y2s, st1, st2):
    p, i = pl.program_id(0), pl.program_id(1)
    _layer_common(p, i, h_ref, a0_ref, a1_ref, wa_ref, ba_ref, ga_ref,
                  bea_ref, wb_ref, bb_ref, y1s, y2s, st1, st2)

    @pl.when(p == 2)
    def _():
        a, cc = _bn_affine(st2[...], gb_ref[...], beb_ref[...])
        out_ref[...] = jnp.maximum(y2s[pl.ds(i * _BLK, _BLK), :] * a + cc, 0.0)


def _layer_last_body(h_ref, a0_ref, a1_ref, wa_ref, ba_ref, ga_ref, bea_ref,
                     wb_ref, bb_ref, gb_ref, beb_ref, bt_ref, lw_ref, lb_ref,
                     out_ref, y1s, y2s, st1, st2, pacc):
    p, i = pl.program_id(0), pl.program_id(1)
    _layer_common(p, i, h_ref, a0_ref, a1_ref, wa_ref, ba_ref, ga_ref,
                  bea_ref, wb_ref, bb_ref, y1s, y2s, st1, st2)

    @pl.when(p == 2)
    def _():
        a, cc = _bn_affine(st2[...], gb_ref[...], beb_ref[...])
        h = jnp.maximum(y2s[pl.ds(i * _BLK, _BLK), :] * a + cc, 0.0)
        bt = bt_ref[0, 0, :]
        onehot = (bt[:, None] == lax.broadcasted_iota(jnp.int32, (_BLK, _G), 1)
                  ).astype(jnp.float32)
        pp = lax.dot_general(onehot, h, (((0,), (0,)), ((), ())),
                             preferred_element_type=jnp.float32)

        @pl.when(i == 0)
        def _():
            pacc[...] = pp

        @pl.when(i > 0)
        def _():
            pacc[...] += pp

        @pl.when(i == _NBLK - 1)
        def _():
            out_ref[...] = (jnp.dot(pacc[...], lw_ref[...],
                                    preferred_element_type=jnp.float32)
                            + lb_ref[...])


def _phase0_blk(p, i):
    return (jnp.where(p == 0, i, _NBLK - 1), 0)


_lay_in_specs = [
    pl.BlockSpec((_BLK, _D), _phase0_blk),
    pl.BlockSpec((1, _BLK, _D), lambda p, i: (0,) + _phase0_blk(p, i)),
    pl.BlockSpec((1, _BLK, _D), lambda p, i: (1,) + _phase0_blk(p, i)),
    pl.BlockSpec((_D, _D), lambda p, i: (0, 0)),
    pl.BlockSpec((1, _D), lambda p, i: (0, 0)),
    pl.BlockSpec((1, _D), lambda p, i: (0, 0)),
    pl.BlockSpec((1, _D), lambda p, i: (0, 0)),
    pl.BlockSpec((_D, _D), lambda p, i: (0, 0)),
    pl.BlockSpec((1, _D), lambda p, i: (0, 0)),
    pl.BlockSpec((1, _D), lambda p, i: (0, 0)),
    pl.BlockSpec((1, _D), lambda p, i: (0, 0)),
]

_lay_scratch = [
    pltpu.VMEM((_N, _D), jnp.float32),
    pltpu.VMEM((_N, _D), jnp.float32),
    pltpu.VMEM((8, _D), jnp.float32),
    pltpu.VMEM((8, _D), jnp.float32),
]

_layer = pl.pallas_call(
    _layer_body,
    grid=(3, _NBLK),
    in_specs=_lay_in_specs,
    out_specs=pl.BlockSpec((_BLK, _D), lambda p, i: (jnp.where(p == 2, i, 0), 0)),
    out_shape=jax.ShapeDtypeStruct((_N, _D), jnp.float32),
    scratch_shapes=_lay_scratch,
)

_layer_last = pl.pallas_call(
    _layer_last_body,
    grid=(3, _NBLK),
    in_specs=_lay_in_specs + [
        pl.BlockSpec((1, 1, _BLK), lambda p, i: (jnp.where(p == 2, i, _NBLK - 1), 0, 0)),
        pl.BlockSpec((_D, 2 * _D), lambda p, i: (0, 0)),
        pl.BlockSpec((1, 2 * _D), lambda p, i: (0, 0)),
    ],
    out_specs=pl.BlockSpec((_G, 2 * _D), lambda p, i: (0, 0)),
    out_shape=jax.ShapeDtypeStruct((_G, 2 * _D), jnp.float32),
    scratch_shapes=_lay_scratch + [pltpu.VMEM((_G, _D), jnp.float32)],
)


def kernel(x, edge_index, batch,
           W0a, b0a, g0a, be0a, W0b, b0b, g0b, be0b,
           W1a, b1a, g1a, be1a, W1b, b1b, g1b, be1b,
           W2a, b2a, g2a, be2a, W2b, b2b, g2b, be2b,
           linW, linb):
    src3 = edge_index[0].astype(jnp.int32).reshape(_NW, _T, _K)
    dst3 = edge_index[1].astype(jnp.int32).reshape(_NW, _T, _K)
    bt3 = batch.astype(jnp.int32).reshape(_NBLK, 1, _BLK)

    p = {
        "W0a": W0a, "b0a": b0a, "g0a": g0a, "be0a": be0a,
        "W0b": W0b, "b0b": b0b, "g0b": g0b, "be0b": be0b,
        "W1a": W1a, "b1a": b1a, "g1a": g1a, "be1a": be1a,
        "W1b": W1b, "b1b": b1b, "g1b": g1b, "be1b": be1b,
        "W2a": W2a, "b2a": b2a, "g2a": g2a, "be2a": be2a,
        "W2b": W2b, "b2b": b2b, "g2b": g2b, "be2b": be2b,
    }

    def row(v):
        return v.reshape(1, -1)

    zc = jnp.zeros((_RPT, _D), jnp.float32)
    h = x
    for l in range(3):
        agg = _sc_agg(h, src3, dst3, zc)
        args = (h, agg, agg, p[f"W{l}a"], row(p[f"b{l}a"]),
                row(p[f"g{l}a"]), row(p[f"be{l}a"]),
                p[f"W{l}b"], row(p[f"b{l}b"]),
                row(p[f"g{l}b"]), row(p[f"be{l}b"]))
        if l < 2:
            h = _layer(*args)
        else:
            out = _layer_last(*args, bt3, linW, row(linb))
    return out


# speedup vs baseline: 1.0653x; 1.0088x over previous
import functools

import jax
import jax.numpy as jnp
from jax import lax
from jax.experimental import pallas as pl
from jax.experimental.pallas import tpu as pltpu
from jax.experimental.pallas import tpu_sc as plsc

_N = 10000
_E = 320000
_D = 128
_G = 128

_NC = 2
_NS = 16
_NW = _NC * _NS
_EW = _E // _NW
_K = 80
_T = _EW // _K
_RPT = 632
_NP = _NS * _RPT

_BLK = 2000
_NBLK = _N // _BLK


def _sc_agg_body(x_hbm, src_hbm, dst_hbm, zeros_hbm, out_hbm,
                 acc, sidx, di0, di1, r0, r1, gs0, gs1, ss0, ss1, is0, is1,
                 zsem):
    r = [r0, r1]
    di = [di0, di1]
    gs = [gs0, gs1]
    ss = [ss0, ss1]
    isem = [is0, is1]
    c = lax.axis_index("c")
    s = lax.axis_index("s")
    w = s * _NC + c

    pltpu.async_copy(zeros_hbm, acc.at[pl.ds(s * _RPT, _RPT)], zsem)
    pltpu.sync_copy(src_hbm.at[w], sidx)

    def didx_start(t, j):
        pltpu.async_copy(dst_hbm.at[w, t], di[j], isem[j])

    def didx_wait(t, j):
        pltpu.make_async_copy(dst_hbm.at[w, t], di[j], isem[j]).wait()

    def gather_start(t, j):
        pltpu.async_copy(x_hbm.at[sidx.at[t]], r[j], gs[j])

    def gather_wait(t, j):
        pltpu.make_async_copy(x_hbm.at[sidx.at[t]], r[j], gs[j]).wait()

    def scatter_start(j):
        pltpu.async_copy(r[j], acc.at[di[j]], ss[j], add=True)

    def scatter_wait(j):
        pltpu.make_async_copy(r[j], acc.at[di[j]], ss[j]).wait()

    def _start(i, _):
        didx_start(i, 0)
        gather_start(i, 0)
        didx_start(i + 1, 1)
        gather_start(i + 1, 1)
        pltpu.make_async_copy(zeros_hbm, acc.at[pl.ds(s * _RPT, _RPT)],
                              zsem).wait()
        plsc.subcore_barrier()
        didx_wait(i, 0)
        gather_wait(i, 0)
        scatter_start(0)
        return 0
    lax.fori_loop(0, jnp.int32(1), _start, 0)

    def _pair(i, _):
        t = 2 * i + 1
        gather_wait(t, 1)
        didx_wait(t, 1)
        scatter_start(1)
        scatter_wait(0)
        didx_start(t + 1, 0)
        gather_start(t + 1, 0)
        gather_wait(t + 1, 0)
        didx_wait(t + 1, 0)
        scatter_start(0)
        scatter_wait(1)

        @pl.when(t + 2 < _T)
        def _():
            didx_start(t + 2, 1)
            gather_start(t + 2, 1)
        return 0
    lax.fori_loop(0, (_T - 1) // 2, _pair, 0)

    scatter_wait(0)

    plsc.subcore_barrier()

    pltpu.sync_copy(acc.at[pl.ds(s * _RPT, _RPT)],
                    out_hbm.at[c, pl.ds(s * _RPT, _RPT)])


_sc_agg = functools.partial(
    pl.kernel,
    out_type=jax.ShapeDtypeStruct((_NC, _NP, _D), jnp.float32),
    mesh=plsc.VectorSubcoreMesh(core_axis_name="c", subcore_axis_name="s"),
    scratch_types=[
        pltpu.VMEM_SHARED((_NP, _D), jnp.float32),
        pltpu.VMEM((_T, _K), jnp.int32),
        pltpu.VMEM((_K,), jnp.int32),
        pltpu.VMEM((_K,), jnp.int32),
        pltpu.VMEM((_K, _D), jnp.float32),
        pltpu.VMEM((_K, _D), jnp.float32),
        pltpu.SemaphoreType.DMA,
        pltpu.SemaphoreType.DMA,
        pltpu.SemaphoreType.DMA,
        pltpu.SemaphoreType.DMA,
        pltpu.SemaphoreType.DMA,
        pltpu.SemaphoreType.DMA,
        pltpu.SemaphoreType.DMA,
    ],
)(_sc_agg_body)


def _bn_affine(st, g, be):
    m = st[0:1, :] * (1.0 / _N)
    v = st[1:2, :] * (1.0 / _N) - m * m
    a = lax.rsqrt(v + 1e-5) * g
    return a, be - m * a


def _acc_stats(i, y, st):
    s1 = jnp.sum(y, axis=0, keepdims=True)
    s2 = jnp.sum(y * y, axis=0, keepdims=True)
    upd = jnp.concatenate([s1, s2, jnp.zeros((6, _D), jnp.float32)], axis=0)

    @pl.when(i == 0)
    def _():
        st[...] = upd

    @pl.when(i > 0)
    def _():
        st[...] += upd


def _layer_common(p, i, h_ref, a0_ref, a1_ref, wa_ref, ba_ref, ga_ref,
                  bea_ref, wb_ref, bb_ref, y1s, y2s, st1, st2):
    @pl.when(p == 0)
    def _():
        u = h_ref[...] + a0_ref[0] + a1_ref[0]
        y = jnp.dot(u, wa_ref[...], preferred_element_type=jnp.float32) + ba_ref[...]
        y1s[pl.ds(i * _BLK, _BLK), :] = y
        _acc_stats(i, y, st1)

    @pl.when(p == 1)
    def _():
        a, cc = _bn_affine(st1[...], ga_ref[...], bea_ref[...])
        z = jnp.maximum(y1s[pl.ds(i * _BLK, _BLK), :] * a + cc, 0.0)
        y2 = jnp.dot(z, wb_ref[...], preferred_element_type=jnp.float32) + bb_ref[...]
        y2s[pl.ds(i * _BLK, _BLK), :] = y2
        _acc_stats(i, y2, st2)


def _layer_body(h_ref, a0_ref, a1_ref, wa_ref, ba_ref, ga_ref, bea_ref,
                wb_ref, bb_ref, gb_ref, beb_ref, out_ref, y1s, y2s, st1, st2):
    p, i = pl.program_id(0), pl.program_id(1)
    _layer_common(p, i, h_ref, a0_ref, a1_ref, wa_ref, ba_ref, ga_ref,
                  bea_ref, wb_ref, bb_ref, y1s, y2s, st1, st2)

    @pl.when(p == 2)
    def _():
        a, cc = _bn_affine(st2[...], gb_ref[...], beb_ref[...])
        out_ref[...] = jnp.maximum(y2s[pl.ds(i * _BLK, _BLK), :] * a + cc, 0.0)


def _layer_last_body(h_ref, a0_ref, a1_ref, wa_ref, ba_ref, ga_ref, bea_ref,
                     wb_ref, bb_ref, gb_ref, beb_ref, bt_ref, lw_ref, lb_ref,
                     out_ref, y1s, y2s, st1, st2, pacc):
    p, i = pl.program_id(0), pl.program_id(1)
    _layer_common(p, i, h_ref, a0_ref, a1_ref, wa_ref, ba_ref, ga_ref,
                  bea_ref, wb_ref, bb_ref, y1s, y2s, st1, st2)

    @pl.when(p == 2)
    def _():
        a, cc = _bn_affine(st2[...], gb_ref[...], beb_ref[...])
        h = jnp.maximum(y2s[pl.ds(i * _BLK, _BLK), :] * a + cc, 0.0)
        bt = bt_ref[0, 0, :]
        onehot = (bt[:, None] == lax.broadcasted_iota(jnp.int32, (_BLK, _G), 1)
                  ).astype(jnp.float32)
        pp = lax.dot_general(onehot, h, (((0,), (0,)), ((), ())),
                             preferred_element_type=jnp.float32)

        @pl.when(i == 0)
        def _():
            pacc[...] = pp

        @pl.when(i > 0)
        def _():
            pacc[...] += pp

        @pl.when(i == _NBLK - 1)
        def _():
            out_ref[...] = (jnp.dot(pacc[...], lw_ref[...],
                                    preferred_element_type=jnp.float32)
                            + lb_ref[...])


def _phase0_blk(p, i):
    return (jnp.where(p == 0, i, _NBLK - 1), 0)


_lay_in_specs = [
    pl.BlockSpec((_BLK, _D), _phase0_blk),
    pl.BlockSpec((1, _BLK, _D), lambda p, i: (0,) + _phase0_blk(p, i)),
    pl.BlockSpec((1, _BLK, _D), lambda p, i: (1,) + _phase0_blk(p, i)),
    pl.BlockSpec((_D, _D), lambda p, i: (0, 0)),
    pl.BlockSpec((1, _D), lambda p, i: (0, 0)),
    pl.BlockSpec((1, _D), lambda p, i: (0, 0)),
    pl.BlockSpec((1, _D), lambda p, i: (0, 0)),
    pl.BlockSpec((_D, _D), lambda p, i: (0, 0)),
    pl.BlockSpec((1, _D), lambda p, i: (0, 0)),
    pl.BlockSpec((1, _D), lambda p, i: (0, 0)),
    pl.BlockSpec((1, _D), lambda p, i: (0, 0)),
]

_lay_scratch = [
    pltpu.VMEM((_N, _D), jnp.float32),
    pltpu.VMEM((_N, _D), jnp.float32),
    pltpu.VMEM((8, _D), jnp.float32),
    pltpu.VMEM((8, _D), jnp.float32),
]

_layer = pl.pallas_call(
    _layer_body,
    grid=(3, _NBLK),
    in_specs=_lay_in_specs,
    out_specs=pl.BlockSpec((_BLK, _D), lambda p, i: (jnp.where(p == 2, i, 0), 0)),
    out_shape=jax.ShapeDtypeStruct((_N, _D), jnp.float32),
    scratch_shapes=_lay_scratch,
)

_layer_last = pl.pallas_call(
    _layer_last_body,
    grid=(3, _NBLK),
    in_specs=_lay_in_specs + [
        pl.BlockSpec((1, 1, _BLK), lambda p, i: (jnp.where(p == 2, i, _NBLK - 1), 0, 0)),
        pl.BlockSpec((_D, 2 * _D), lambda p, i: (0, 0)),
        pl.BlockSpec((1, 2 * _D), lambda p, i: (0, 0)),
    ],
    out_specs=pl.BlockSpec((_G, 2 * _D), lambda p, i: (0, 0)),
    out_shape=jax.ShapeDtypeStruct((_G, 2 * _D), jnp.float32),
    scratch_shapes=_lay_scratch + [pltpu.VMEM((_G, _D), jnp.float32)],
)


def kernel(x, edge_index, batch,
           W0a, b0a, g0a, be0a, W0b, b0b, g0b, be0b,
           W1a, b1a, g1a, be1a, W1b, b1b, g1b, be1b,
           W2a, b2a, g2a, be2a, W2b, b2b, g2b, be2b,
           linW, linb):
    src3 = edge_index[0].astype(jnp.int32).reshape(_NW, _T, _K)
    dst3 = edge_index[1].astype(jnp.int32).reshape(_NW, _T, _K)
    bt3 = batch.astype(jnp.int32).reshape(_NBLK, 1, _BLK)

    p = {
        "W0a": W0a, "b0a": b0a, "g0a": g0a, "be0a": be0a,
        "W0b": W0b, "b0b": b0b, "g0b": g0b, "be0b": be0b,
        "W1a": W1a, "b1a": b1a, "g1a": g1a, "be1a": be1a,
        "W1b": W1b, "b1b": b1b, "g1b": g1b, "be1b": be1b,
        "W2a": W2a, "b2a": b2a, "g2a": g2a, "be2a": be2a,
        "W2b": W2b, "b2b": b2b, "g2b": g2b, "be2b": be2b,
    }

    def row(v):
        return v.reshape(1, -1)

    zc = jnp.zeros((_RPT, _D), jnp.float32)
    h = x
    for l in range(3):
        agg = _sc_agg(h, src3, dst3, zc)
        args = (h, agg, agg, p[f"W{l}a"], row(p[f"b{l}a"]),
                row(p[f"g{l}a"]), row(p[f"be{l}a"]),
                p[f"W{l}b"], row(p[f"b{l}b"]),
                row(p[f"g{l}b"]), row(p[f"be{l}b"]))
        if l < 2:
            h = _layer(*args)
        else:
            out = _layer_last(*args, bt3, linW, row(linb))
    return out
